# Initial kernel scaffold; baseline (speedup 1.0000x reference)
import jax
import jax.numpy as jnp
from jax.experimental import pallas as pl


def kernel(G_p, Wb, We, k):
    raise NotImplementedError("write your pallas kernel here")


# speedup vs baseline: 332.2299x; 332.2299x over previous
import functools

import jax
import jax.numpy as jnp
from jax import lax
from jax.experimental import pallas as pl
from jax.experimental.pallas import tpu as pltpu
from jax.experimental.pallas import tpu_sc as plsc

S = 8192
TOPK = 128
M = 160
L = 16
NCH = S // L
NB_M = M // L
NB_K = TOPK // L
FLAT_PAD = 2**30
NEG_INF = float("-inf")


def _iota16():
    return lax.iota(jnp.int32, 16)


_GDN = lax.GatherDimensionNumbers(
    offset_dims=(), collapsed_slice_dims=(0,), start_index_map=(0,))


def _perm(x, idx):
    return lax.gather(x, idx[:, None], _GDN, (1,),
                      mode=lax.GatherScatterMode.PROMISE_IN_BOUNDS)


def _before(k1, v1, k2, v2):
    return (k1 > k2) | ((k1 == k2) & (v1 <= v2))


def _cmpx(kk, vv, dist, desc_mask):
    idx = _iota16() ^ dist
    pk = _perm(kk, idx)
    pv = _perm(vv, idx)
    first = (_iota16() & dist) == 0
    win = _before(kk, vv, pk, pv)
    keep = win == (first == desc_mask)
    return jnp.where(keep, kk, pk), jnp.where(keep, vv, pv)


def _bmerge16(kk, vv):
    for dist in (8, 4, 2, 1):
        kk, vv = _cmpx(kk, vv, dist, True)
    return kk, vv


def _sort16(kk, vv):
    io = _iota16()
    for blk in (2, 4, 8, 16):
        desc_mask = (io & blk) == 0
        dist = blk // 2
        while dist >= 1:
            kk, vv = _cmpx(kk, vv, dist, desc_mask)
            dist //= 2
    return kk, vv


def _merge2x16(ak, av, bk, bv):
    rbk = jnp.flip(bk, 0)
    rbv = jnp.flip(bv, 0)
    take = _before(ak, av, rbk, rbv)
    hk = jnp.where(take, ak, rbk)
    hv = jnp.where(take, av, rbv)
    lk = jnp.where(take, rbk, ak)
    lv = jnp.where(take, rbv, av)
    hk, hv = _bmerge16(hk, hv)
    lk, lv = _bmerge16(lk, lv)
    return hk, hv, lk, lv


def _insert(bufs, ck, cv):
    out = []
    for bk, bv in bufs:
        hk, hv, ck, cv = _merge2x16(bk, bv, ck, cv)
        out.append((hk, hv))
    return out


def _flatten(bufs):
    return tuple(x for kv in bufs for x in kv)


def _unflatten(flat):
    return [(flat[2 * i], flat[2 * i + 1]) for i in range(len(flat) // 2)]


def _topm_scan(src_ref, nbuf, nchunks):
    def body(i, carry):
        bufs = _unflatten(carry)
        ck = src_ref[pl.ds(i * L, L)]
        tau = jnp.min(bufs[-1][0])
        cmax = jnp.max(ck)

        def ins(c):
            sk, sv = _sort16(ck, i * L + _iota16())
            return _flatten(_insert(_unflatten(c), sk, sv))

        return lax.cond(cmax >= tau, ins, lambda c: c, carry)

    init = _flatten([(jnp.full((L,), NEG_INF, jnp.float32),
                      jnp.full((L,), FLAT_PAD, jnp.int32)) for _ in range(nbuf)])
    return _unflatten(lax.fori_loop(0, nchunks, body, init))


def _sc_body(b_hbm, e_hbm, oi_hbm, oj_hbm, ov_hbm,
             bv, ev, cv, dv, biv, iiv, ejv, jjv, oiv, ojv, ovv):
    wid = lax.axis_index("s") * 2 + lax.axis_index("c")

    @pl.when(wid == 0)
    def _():
        pltpu.sync_copy(b_hbm, bv)
        pltpu.sync_copy(e_hbm, ev)

        def fwd(i, carry):
            pmax, seb, see = carry
            bx = bv[pl.ds(i * L, L)]
            ey = ev[pl.ds(i * L, L)]
            pm = jnp.maximum(plsc.cummax(bx), pmax)
            cv[pl.ds(i * L, L)] = pm + ey
            return (jnp.max(pm), seb + jnp.exp(bx), see + jnp.exp(ey))

        pmax, seb, see = lax.fori_loop(
            0, NCH, fwd, (NEG_INF, jnp.zeros((L,), jnp.float32),
                          jnp.zeros((L,), jnp.float32)))
        denom = jnp.sum(seb) * jnp.sum(see)

        def bwd(t, smax):
            i = NCH - 1 - t
            ey = ev[pl.ds(i * L, L)]
            sm = jnp.maximum(jnp.flip(plsc.cummax(jnp.flip(ey, 0)), 0), smax)
            dv[pl.ds(i * L, L)] = bv[pl.ds(i * L, L)] + sm
            return jnp.max(sm)

        lax.fori_loop(0, NCH, bwd, NEG_INF)

        jbufs = _topm_scan(cv, NB_M, NCH)
        ibufs = _topm_scan(dv, NB_M, NCH)

        for t in range(NB_M):
            ji = jbufs[t][1]
            jjv[pl.ds(t * L, L)] = ji
            ejv[pl.ds(t * L, L)] = plsc.load_gather(ev, [ji])
            ii = ibufs[t][1]
            iiv[pl.ds(t * L, L)] = ii
            biv[pl.ds(t * L, L)] = plsc.load_gather(bv, [ii])

        def frow(r, carry):
            bvec = plsc.load_gather(biv, [jnp.full((L,), r, jnp.int32)])
            ivec = plsc.load_gather(iiv, [jnp.full((L,), r, jnp.int32)])
            for t in range(NB_M):
                ek = ejv[pl.ds(t * L, L)]
                jv = jjv[pl.ds(t * L, L)]
                key = jnp.where(jv >= ivec, bvec + ek, NEG_INF)
                flat = ivec * S + jv
                bufs = _unflatten(carry)
                tau = jnp.min(bufs[-1][0])
                cmax = jnp.max(key)

                def ins(c, key=key, flat=flat):
                    sk, sv = _sort16(key, flat)
                    return _flatten(_insert(_unflatten(c), sk, sv))

                carry = lax.cond(cmax >= tau, ins, lambda c: c, carry)
            return carry

        finit = _flatten([(jnp.full((L,), NEG_INF, jnp.float32),
                           jnp.full((L,), FLAT_PAD, jnp.int32)) for _ in range(NB_K)])
        fbufs = _unflatten(lax.fori_loop(0, M, frow, finit))

        for t in range(NB_K):
            fk, fv = fbufs[t]
            oiv[pl.ds(t * L, L)] = lax.shift_right_logical(fv, 13)
            ojv[pl.ds(t * L, L)] = fv & (S - 1)
            ovv[pl.ds(t * L, L)] = jnp.exp(fk) / denom

        pltpu.sync_copy(oiv, oi_hbm)
        pltpu.sync_copy(ojv, oj_hbm)
        pltpu.sync_copy(ovv, ov_hbm)


@jax.jit
def _sc_select(b, e):
    mesh = plsc.VectorSubcoreMesh(core_axis_name="c", subcore_axis_name="s")
    fn = functools.partial(
        pl.kernel,
        mesh=mesh,
        compiler_params=pltpu.CompilerParams(needs_layout_passes=False),
        out_type=[
            jax.ShapeDtypeStruct((TOPK,), jnp.int32),
            jax.ShapeDtypeStruct((TOPK,), jnp.int32),
            jax.ShapeDtypeStruct((TOPK,), jnp.float32),
        ],
        scratch_types=[
            pltpu.VMEM((S,), jnp.float32),
            pltpu.VMEM((S,), jnp.float32),
            pltpu.VMEM((S,), jnp.float32),
            pltpu.VMEM((S,), jnp.float32),
            pltpu.VMEM((M,), jnp.float32),
            pltpu.VMEM((M,), jnp.int32),
            pltpu.VMEM((M,), jnp.float32),
            pltpu.VMEM((M,), jnp.int32),
            pltpu.VMEM((TOPK,), jnp.int32),
            pltpu.VMEM((TOPK,), jnp.int32),
            pltpu.VMEM((TOPK,), jnp.float32),
        ],
    )(_sc_body)
    return fn(b, e)


def _tc_matvec(G, W):
    def body(g_ref, w_ref, o_ref):
        o_ref[...] = jnp.dot(g_ref[...], w_ref[...],
                             preferred_element_type=jnp.float32)

    return pl.pallas_call(
        body,
        out_shape=jax.ShapeDtypeStruct((S, 2), jnp.float32),
    )(G, W)


def kernel(G_p, Wb, We, k):
    del k
    be = _tc_matvec(G_p, jnp.concatenate([Wb, We], axis=1))
    b = be[:, 0]
    e = be[:, 1]
    oi, oj, vals = _sc_select(b, e)
    return (jnp.concatenate([oi[:, None], oj[:, None]], axis=1), vals)


# speedup vs baseline: 583.4873x; 1.7563x over previous
import functools

import jax
import jax.numpy as jnp
from jax import lax
from jax.experimental import pallas as pl
from jax.experimental.pallas import tpu as pltpu
from jax.experimental.pallas import tpu_sc as plsc

S = 8192
TOPK = 128
M = 160
L = 16
NCH = S // L
NB_M = M // L
NB_K = TOPK // L
FLAT_PAD = 2**30
NEG_INF = float("-inf")


def _iota16():
    return lax.iota(jnp.int32, 16)


_GDN = lax.GatherDimensionNumbers(
    offset_dims=(), collapsed_slice_dims=(0,), start_index_map=(0,))


def _perm(x, idx):
    return lax.gather(x, idx[:, None], _GDN, (1,),
                      mode=lax.GatherScatterMode.PROMISE_IN_BOUNDS)


def _before(k1, v1, k2, v2):
    return (k1 > k2) | ((k1 == k2) & (v1 <= v2))


def _cmpx(kk, vv, dist, desc_mask):
    idx = _iota16() ^ dist
    pk = _perm(kk, idx)
    pv = _perm(vv, idx)
    first = (_iota16() & dist) == 0
    win = _before(kk, vv, pk, pv)
    keep = win == (first == desc_mask)
    return jnp.where(keep, kk, pk), jnp.where(keep, vv, pv)


def _bmerge16(kk, vv):
    for dist in (8, 4, 2, 1):
        kk, vv = _cmpx(kk, vv, dist, True)
    return kk, vv


def _sort16(kk, vv):
    io = _iota16()
    for blk in (2, 4, 8, 16):
        desc_mask = (io & blk) == 0
        dist = blk // 2
        while dist >= 1:
            kk, vv = _cmpx(kk, vv, dist, desc_mask)
            dist //= 2
    return kk, vv


def _merge2x16(ak, av, bk, bv):
    rbk = jnp.flip(bk, 0)
    rbv = jnp.flip(bv, 0)
    take = _before(ak, av, rbk, rbv)
    hk = jnp.where(take, ak, rbk)
    hv = jnp.where(take, av, rbv)
    lk = jnp.where(take, rbk, ak)
    lv = jnp.where(take, rbv, av)
    hk, hv = _bmerge16(hk, hv)
    lk, lv = _bmerge16(lk, lv)
    return hk, hv, lk, lv


def _merge2x16_hw(ak, av, bk, bv):
    rbk = jnp.flip(bk, 0)
    rbv = jnp.flip(bv, 0)
    take = ak >= rbk
    hk = jnp.where(take, ak, rbk)
    hv = jnp.where(take, av, rbv)
    lk = jnp.where(take, rbk, ak)
    lv = jnp.where(take, rbv, av)
    hk, hv = plsc.sort_key_val(hk, hv, descending=True)
    lk, lv = plsc.sort_key_val(lk, lv, descending=True)
    return hk, hv, lk, lv


def _insert(bufs, ck, cv, hw=False):
    merge = _merge2x16_hw if hw else _merge2x16
    out = []
    for bk, bv in bufs:
        hk, hv, ck, cv = merge(bk, bv, ck, cv)
        out.append((hk, hv))
    return out


def _flatten(bufs):
    return tuple(x for kv in bufs for x in kv)


def _unflatten(flat):
    return [(flat[2 * i], flat[2 * i + 1]) for i in range(len(flat) // 2)]


def _topm_scan(src_ref, nbuf, nchunks):
    def body(i, carry):
        bufs = _unflatten(carry)
        ck = src_ref[pl.ds(i * L, L)]
        tau = jnp.min(bufs[-1][0])
        cmax = jnp.max(ck)

        def ins(c):
            sk, sv = plsc.sort_key_val(ck, i * L + _iota16(), descending=True)
            return _flatten(_insert(_unflatten(c), sk, sv, hw=True))

        return lax.cond(cmax >= tau, ins, lambda c: c, carry)

    init = _flatten([(jnp.full((L,), NEG_INF, jnp.float32),
                      jnp.full((L,), FLAT_PAD, jnp.int32)) for _ in range(nbuf)])
    return _unflatten(lax.fori_loop(0, nchunks, body, init))


def _sc_body(b_hbm, e_hbm, oi_hbm, oj_hbm, ov_hbm,
             bv, ev, cv, dv, biv, iiv, ejv, jjv, oiv, ojv, ovv):
    wid = lax.axis_index("s") * 2 + lax.axis_index("c")

    @pl.when(wid == 0)
    def _():
        pltpu.sync_copy(b_hbm, bv)
        pltpu.sync_copy(e_hbm, ev)

        def fwd(i, carry):
            pmax, seb, see = carry
            bx = bv[pl.ds(i * L, L)]
            ey = ev[pl.ds(i * L, L)]
            pm = jnp.maximum(plsc.cummax(bx), pmax)
            cv[pl.ds(i * L, L)] = pm + ey
            return (jnp.max(pm), seb + jnp.exp(bx), see + jnp.exp(ey))

        pmax, seb, see = lax.fori_loop(
            0, NCH, fwd, (NEG_INF, jnp.zeros((L,), jnp.float32),
                          jnp.zeros((L,), jnp.float32)))
        denom = jnp.sum(seb) * jnp.sum(see)

        def bwd(t, smax):
            i = NCH - 1 - t
            ey = ev[pl.ds(i * L, L)]
            sm = jnp.maximum(jnp.flip(plsc.cummax(jnp.flip(ey, 0)), 0), smax)
            dv[pl.ds(i * L, L)] = bv[pl.ds(i * L, L)] + sm
            return jnp.max(sm)

        lax.fori_loop(0, NCH, bwd, NEG_INF)

        jbufs = _topm_scan(cv, NB_M, NCH)
        ibufs = _topm_scan(dv, NB_M, NCH)

        for t in range(NB_M):
            ji = jbufs[t][1]
            jjv[pl.ds(t * L, L)] = ji
            ejv[pl.ds(t * L, L)] = plsc.load_gather(ev, [ji])
            ii = ibufs[t][1]
            iiv[pl.ds(t * L, L)] = ii
            biv[pl.ds(t * L, L)] = plsc.load_gather(bv, [ii])

        emax = NEG_INF
        for t in range(NB_M):
            emax = jnp.maximum(emax, jnp.max(ejv[pl.ds(t * L, L)]))

        def frow(r, carry):
            bufs = _unflatten(carry)
            tau0 = jnp.min(bufs[-1][0])
            bvec = plsc.load_gather(biv, [jnp.full((L,), r, jnp.int32)])

            def do_row(carry):
                ivec = plsc.load_gather(iiv, [jnp.full((L,), r, jnp.int32)])
                for t in range(NB_M):
                    ek = ejv[pl.ds(t * L, L)]
                    jv = jjv[pl.ds(t * L, L)]
                    key = jnp.where(jv >= ivec, bvec + ek, NEG_INF)
                    flat = ivec * S + jv
                    bufs = _unflatten(carry)
                    tau = jnp.min(bufs[-1][0])
                    cmax = jnp.max(key)

                    def ins(c, key=key, flat=flat):
                        sk, sv = _sort16(key, flat)
                        return _flatten(_insert(_unflatten(c), sk, sv))

                    carry = lax.cond(cmax >= tau, ins, lambda c: c, carry)
                return carry

            return lax.cond(jnp.max(bvec) + emax >= tau0, do_row,
                            lambda c: c, carry)

        finit = _flatten([(jnp.full((L,), NEG_INF, jnp.float32),
                           jnp.full((L,), FLAT_PAD, jnp.int32)) for _ in range(NB_K)])
        fbufs = _unflatten(lax.fori_loop(0, M, frow, finit))

        for t in range(NB_K):
            fk, fv = fbufs[t]
            oiv[pl.ds(t * L, L)] = lax.shift_right_logical(fv, 13)
            ojv[pl.ds(t * L, L)] = fv & (S - 1)
            ovv[pl.ds(t * L, L)] = jnp.exp(fk) / denom

        pltpu.sync_copy(oiv, oi_hbm)
        pltpu.sync_copy(ojv, oj_hbm)
        pltpu.sync_copy(ovv, ov_hbm)


@jax.jit
def _sc_select(b, e):
    mesh = plsc.VectorSubcoreMesh(core_axis_name="c", subcore_axis_name="s")
    fn = functools.partial(
        pl.kernel,
        mesh=mesh,
        compiler_params=pltpu.CompilerParams(needs_layout_passes=False),
        out_type=[
            jax.ShapeDtypeStruct((TOPK,), jnp.int32),
            jax.ShapeDtypeStruct((TOPK,), jnp.int32),
            jax.ShapeDtypeStruct((TOPK,), jnp.float32),
        ],
        scratch_types=[
            pltpu.VMEM((S,), jnp.float32),
            pltpu.VMEM((S,), jnp.float32),
            pltpu.VMEM((S,), jnp.float32),
            pltpu.VMEM((S,), jnp.float32),
            pltpu.VMEM((M,), jnp.float32),
            pltpu.VMEM((M,), jnp.int32),
            pltpu.VMEM((M,), jnp.float32),
            pltpu.VMEM((M,), jnp.int32),
            pltpu.VMEM((TOPK,), jnp.int32),
            pltpu.VMEM((TOPK,), jnp.int32),
            pltpu.VMEM((TOPK,), jnp.float32),
        ],
    )(_sc_body)
    return fn(b, e)


def _tc_matvec(G, W):
    def body(g_ref, w_ref, o_ref):
        o_ref[...] = jnp.dot(g_ref[...], w_ref[...],
                             preferred_element_type=jnp.float32)

    return pl.pallas_call(
        body,
        out_shape=jax.ShapeDtypeStruct((S, 2), jnp.float32),
    )(G, W)


def kernel(G_p, Wb, We, k):
    del k
    be = _tc_matvec(G_p, jnp.concatenate([Wb, We], axis=1))
    b = be[:, 0]
    e = be[:, 1]
    oi, oj, vals = _sc_select(b, e)
    return (jnp.concatenate([oi[:, None], oj[:, None]], axis=1), vals)


# speedup vs baseline: 918.3321x; 1.5739x over previous
import functools

import jax
import jax.numpy as jnp
from jax import lax
from jax.experimental import pallas as pl
from jax.experimental.pallas import tpu as pltpu
from jax.experimental.pallas import tpu_sc as plsc

S = 8192
TOPK = 128
M = 160
L = 16
NW = 16
SLICE = S // NW
NCW = SLICE // L
NB_M = M // L
NB_K = TOPK // L
ROWS_W = M // NW
FLAT_PAD = 2**30
NEG_INF = float("-inf")


def _iota16():
    return lax.iota(jnp.int32, 16)


_GDN = lax.GatherDimensionNumbers(
    offset_dims=(), collapsed_slice_dims=(0,), start_index_map=(0,))


def _perm(x, idx):
    return lax.gather(x, idx[:, None], _GDN, (1,),
                      mode=lax.GatherScatterMode.PROMISE_IN_BOUNDS)


def _before(k1, v1, k2, v2):
    return (k1 > k2) | ((k1 == k2) & (v1 <= v2))


def _cmpx(kk, vv, dist, desc_mask):
    idx = _iota16() ^ dist
    pk = _perm(kk, idx)
    pv = _perm(vv, idx)
    first = (_iota16() & dist) == 0
    win = _before(kk, vv, pk, pv)
    keep = win == (first == desc_mask)
    return jnp.where(keep, kk, pk), jnp.where(keep, vv, pv)


def _bmerge16(kk, vv):
    for dist in (8, 4, 2, 1):
        kk, vv = _cmpx(kk, vv, dist, True)
    return kk, vv


def _sort16(kk, vv):
    io = _iota16()
    for blk in (2, 4, 8, 16):
        desc_mask = (io & blk) == 0
        dist = blk // 2
        while dist >= 1:
            kk, vv = _cmpx(kk, vv, dist, desc_mask)
            dist //= 2
    return kk, vv


def _merge2x16(ak, av, bk, bv):
    rbk = jnp.flip(bk, 0)
    rbv = jnp.flip(bv, 0)
    take = _before(ak, av, rbk, rbv)
    hk = jnp.where(take, ak, rbk)
    hv = jnp.where(take, av, rbv)
    lk = jnp.where(take, rbk, ak)
    lv = jnp.where(take, rbv, av)
    hk, hv = _bmerge16(hk, hv)
    lk, lv = _bmerge16(lk, lv)
    return hk, hv, lk, lv


def _merge2x16_hw(ak, av, bk, bv):
    rbk = jnp.flip(bk, 0)
    rbv = jnp.flip(bv, 0)
    take = ak >= rbk
    hk = jnp.where(take, ak, rbk)
    hv = jnp.where(take, av, rbv)
    lk = jnp.where(take, rbk, ak)
    lv = jnp.where(take, rbv, av)
    hk, hv = plsc.sort_key_val(hk, hv, descending=True)
    lk, lv = plsc.sort_key_val(lk, lv, descending=True)
    return hk, hv, lk, lv


def _insert(bufs, ck, cv, hw=False):
    merge = _merge2x16_hw if hw else _merge2x16
    out = []
    for bk, bv in bufs:
        hk, hv, ck, cv = merge(bk, bv, ck, cv)
        out.append((hk, hv))
    return out


def _flatten(bufs):
    return tuple(x for kv in bufs for x in kv)


def _unflatten(flat):
    return [(flat[2 * i], flat[2 * i + 1]) for i in range(len(flat) // 2)]


def _init_bufs(nbuf):
    return _flatten([(jnp.full((L,), NEG_INF, jnp.float32),
                      jnp.full((L,), FLAT_PAD, jnp.int32))
                     for _ in range(nbuf)])


def _merge_lists_scan(keys_ref, vals_ref, nbuf, nchunks, hw):
    def body(i, carry):
        bufs = _unflatten(carry)
        ck = keys_ref[pl.ds(i * L, L)]
        cv = vals_ref[pl.ds(i * L, L)]
        tau = jnp.min(bufs[-1][0])
        cmax = jnp.max(ck)

        def ins(c):
            if hw:
                sk, sv = plsc.sort_key_val(ck, cv, descending=True)
            else:
                sk, sv = _sort16(ck, cv)
            return _flatten(_insert(_unflatten(c), sk, sv, hw=hw))

        return lax.cond(cmax >= tau, ins, lambda c: c, carry)

    return _unflatten(lax.fori_loop(0, nchunks, body, _init_bufs(nbuf)))


def _sc_body(b_hbm, e_hbm, oi_hbm, oj_hbm, ov_hbm,
             bfull, efull, stv, st_l, ck_st, ci_st, mk_l, mv_l,
             ej_l, jj_l, bi_l, ii_l, fk_l, fi_l, oiv, ojv, ovv,
             sh_stats, sh_ck, sh_ci, sh_dk, sh_di,
             sh_ej, sh_jj, sh_bi, sh_ii, sh_fk, sh_fi):
    core = lax.axis_index("c")
    sub = lax.axis_index("s")

    @pl.when(core == 0)
    def _():
        w = sub
        base = w * SLICE
        io = _iota16()
        pltpu.sync_copy(b_hbm, bfull)
        pltpu.sync_copy(e_hbm, efull)

        def st(i, carry):
            bmax, emax, seb, see = carry
            bx = bfull[pl.ds(base + i * L, L)]
            ey = efull[pl.ds(base + i * L, L)]
            return (jnp.maximum(bmax, jnp.max(bx)),
                    jnp.maximum(emax, jnp.max(ey)),
                    seb + jnp.exp(bx), see + jnp.exp(ey))

        bmax, emax, seb, see = lax.fori_loop(
            0, NCW, st, (NEG_INF, NEG_INF,
                         jnp.zeros((L,), jnp.float32),
                         jnp.zeros((L,), jnp.float32)))
        statv = jnp.where(io == 0, bmax,
                          jnp.where(io == 1, emax,
                                    jnp.where(io == 2, jnp.sum(seb),
                                              jnp.sum(see))))
        stv[...] = statv
        pltpu.sync_copy(stv, sh_stats.at[pl.ds(w * L, L)])
        plsc.subcore_barrier()

        pltpu.sync_copy(sh_stats, st_l)
        bmaxs = plsc.load_gather(st_l, [io * L])
        emaxs = plsc.load_gather(st_l, [io * L + 1])
        sebs = plsc.load_gather(st_l, [io * L + 2])
        sees = plsc.load_gather(st_l, [io * L + 3])
        pmcarry = jnp.max(jnp.where(io < w, bmaxs, NEG_INF))
        smcarry = jnp.max(jnp.where(io > w, emaxs, NEG_INF))
        denom = jnp.sum(sebs) * jnp.sum(sees)

        def fwd(i, carry):
            pmax = carry[0]
            bufs = _unflatten(carry[1:])
            bx = bfull[pl.ds(base + i * L, L)]
            ey = efull[pl.ds(base + i * L, L)]
            pm = jnp.maximum(plsc.cummax(bx), pmax)
            ck = pm + ey
            tau = jnp.min(bufs[-1][0])
            cmax = jnp.max(ck)

            def ins(c):
                sk, sv = plsc.sort_key_val(ck, base + i * L + io,
                                           descending=True)
                return _flatten(_insert(_unflatten(c), sk, sv, hw=True))

            newbufs = lax.cond(cmax >= tau, ins, lambda c: c, carry[1:])
            return (jnp.max(pm),) + tuple(newbufs)

        cres = lax.fori_loop(0, NCW, fwd, (pmcarry,) + _init_bufs(NB_M))
        jbufs = _unflatten(cres[1:])

        def bwd(t, carry):
            i = NCW - 1 - t
            smax = carry[0]
            bufs = _unflatten(carry[1:])
            bx = bfull[pl.ds(base + i * L, L)]
            ey = efull[pl.ds(base + i * L, L)]
            sm = jnp.maximum(jnp.flip(plsc.cummax(jnp.flip(ey, 0)), 0), smax)
            dk = bx + sm
            tau = jnp.min(bufs[-1][0])
            cmax = jnp.max(dk)

            def ins(c):
                sk, sv = plsc.sort_key_val(dk, base + i * L + io,
                                           descending=True)
                return _flatten(_insert(_unflatten(c), sk, sv, hw=True))

            newbufs = lax.cond(cmax >= tau, ins, lambda c: c, carry[1:])
            return (jnp.max(sm),) + tuple(newbufs)

        dres = lax.fori_loop(0, NCW, bwd, (smcarry,) + _init_bufs(NB_M))
        ibufs = _unflatten(dres[1:])

        for t in range(NB_M):
            ck_st[pl.ds(t * L, L)] = jbufs[t][0]
            ci_st[pl.ds(t * L, L)] = jbufs[t][1]
        pltpu.sync_copy(ck_st, sh_ck.at[pl.ds(w * M, M)])
        pltpu.sync_copy(ci_st, sh_ci.at[pl.ds(w * M, M)])
        for t in range(NB_M):
            ck_st[pl.ds(t * L, L)] = ibufs[t][0]
            ci_st[pl.ds(t * L, L)] = ibufs[t][1]
        pltpu.sync_copy(ck_st, sh_dk.at[pl.ds(w * M, M)])
        pltpu.sync_copy(ci_st, sh_di.at[pl.ds(w * M, M)])
        plsc.subcore_barrier()

        @pl.when(w == 0)
        def _():
            pltpu.sync_copy(sh_ck, mk_l)
            pltpu.sync_copy(sh_ci, mv_l)
            gbufs = _merge_lists_scan(mk_l, mv_l, NB_M, NW * NB_M, hw=True)
            for t in range(NB_M):
                ji = gbufs[t][1]
                jj_l[pl.ds(t * L, L)] = ji
                ej_l[pl.ds(t * L, L)] = plsc.load_gather(efull, [ji])
            pltpu.sync_copy(jj_l, sh_jj)
            pltpu.sync_copy(ej_l, sh_ej)

        @pl.when(w == 1)
        def _():
            pltpu.sync_copy(sh_dk, mk_l)
            pltpu.sync_copy(sh_di, mv_l)
            gbufs = _merge_lists_scan(mk_l, mv_l, NB_M, NW * NB_M, hw=True)
            for t in range(NB_M):
                ii = gbufs[t][1]
                ii_l[pl.ds(t * L, L)] = ii
                bi_l[pl.ds(t * L, L)] = plsc.load_gather(bfull, [ii])
            pltpu.sync_copy(ii_l, sh_ii)
            pltpu.sync_copy(bi_l, sh_bi)

        plsc.subcore_barrier()

        pltpu.sync_copy(sh_ej, ej_l)
        pltpu.sync_copy(sh_jj, jj_l)
        pltpu.sync_copy(sh_bi, bi_l)
        pltpu.sync_copy(sh_ii, ii_l)
        emax_c = NEG_INF
        for t in range(NB_M):
            emax_c = jnp.maximum(emax_c, jnp.max(ej_l[pl.ds(t * L, L)]))

        def frow(ri, carry):
            r = w * ROWS_W + ri
            bufs = _unflatten(carry)
            tau0 = jnp.min(bufs[-1][0])
            bvec = plsc.load_gather(bi_l, [jnp.full((L,), r, jnp.int32)])

            def do_row(carry):
                ivec = plsc.load_gather(ii_l, [jnp.full((L,), r, jnp.int32)])
                for t in range(NB_M):
                    ek = ej_l[pl.ds(t * L, L)]
                    jv = jj_l[pl.ds(t * L, L)]
                    key = jnp.where(jv >= ivec, bvec + ek, NEG_INF)
                    flat = ivec * S + jv
                    bufs = _unflatten(carry)
                    tau = jnp.min(bufs[-1][0])
                    cmax = jnp.max(key)

                    def ins(c, key=key, flat=flat):
                        sk, sv = _sort16(key, flat)
                        return _flatten(_insert(_unflatten(c), sk, sv))

                    carry = lax.cond(cmax >= tau, ins, lambda c: c, carry)
                return carry

            return lax.cond(jnp.max(bvec) + emax_c >= tau0, do_row,
                            lambda c: c, carry)

        fbufs = _unflatten(lax.fori_loop(0, ROWS_W, frow, _init_bufs(NB_K)))
        for t in range(NB_K):
            ovv[pl.ds(t * L, L)] = fbufs[t][0]
            oiv[pl.ds(t * L, L)] = fbufs[t][1]
        pltpu.sync_copy(ovv, sh_fk.at[pl.ds(w * TOPK, TOPK)])
        pltpu.sync_copy(oiv, sh_fi.at[pl.ds(w * TOPK, TOPK)])
        plsc.subcore_barrier()

        @pl.when(w == 0)
        def _():
            pltpu.sync_copy(sh_fk, fk_l)
            pltpu.sync_copy(sh_fi, fi_l)
            gbufs = _merge_lists_scan(fk_l, fi_l, NB_K, NW * NB_K, hw=False)
            for t in range(NB_K):
                fk, fv = gbufs[t]
                oiv[pl.ds(t * L, L)] = lax.shift_right_logical(fv, 13)
                ojv[pl.ds(t * L, L)] = fv & (S - 1)
                ovv[pl.ds(t * L, L)] = jnp.exp(fk) / denom
            pltpu.sync_copy(oiv, oi_hbm)
            pltpu.sync_copy(ojv, oj_hbm)
            pltpu.sync_copy(ovv, ov_hbm)


@jax.jit
def _sc_select(b, e):
    mesh = plsc.VectorSubcoreMesh(core_axis_name="c", subcore_axis_name="s")
    fn = functools.partial(
        pl.kernel,
        mesh=mesh,
        compiler_params=pltpu.CompilerParams(needs_layout_passes=False),
        out_type=[
            jax.ShapeDtypeStruct((TOPK,), jnp.int32),
            jax.ShapeDtypeStruct((TOPK,), jnp.int32),
            jax.ShapeDtypeStruct((TOPK,), jnp.float32),
        ],
        scratch_types=[
            pltpu.VMEM((S,), jnp.float32),
            pltpu.VMEM((S,), jnp.float32),
            pltpu.VMEM((L,), jnp.float32),
            pltpu.VMEM((NW * L,), jnp.float32),
            pltpu.VMEM((M,), jnp.float32),
            pltpu.VMEM((M,), jnp.int32),
            pltpu.VMEM((NW * M,), jnp.float32),
            pltpu.VMEM((NW * M,), jnp.int32),
            pltpu.VMEM((M,), jnp.float32),
            pltpu.VMEM((M,), jnp.int32),
            pltpu.VMEM((M,), jnp.float32),
            pltpu.VMEM((M,), jnp.int32),
            pltpu.VMEM((NW * TOPK,), jnp.float32),
            pltpu.VMEM((NW * TOPK,), jnp.int32),
            pltpu.VMEM((TOPK,), jnp.int32),
            pltpu.VMEM((TOPK,), jnp.int32),
            pltpu.VMEM((TOPK,), jnp.float32),
            pltpu.VMEM_SHARED((NW * L,), jnp.float32),
            pltpu.VMEM_SHARED((NW * M,), jnp.float32),
            pltpu.VMEM_SHARED((NW * M,), jnp.int32),
            pltpu.VMEM_SHARED((NW * M,), jnp.float32),
            pltpu.VMEM_SHARED((NW * M,), jnp.int32),
            pltpu.VMEM_SHARED((M,), jnp.float32),
            pltpu.VMEM_SHARED((M,), jnp.int32),
            pltpu.VMEM_SHARED((M,), jnp.float32),
            pltpu.VMEM_SHARED((M,), jnp.int32),
            pltpu.VMEM_SHARED((NW * TOPK,), jnp.float32),
            pltpu.VMEM_SHARED((NW * TOPK,), jnp.int32),
        ],
    )(_sc_body)
    return fn(b, e)


def _tc_matvec(G, W):
    def body(g_ref, w_ref, o_ref):
        o_ref[...] = jnp.dot(g_ref[...], w_ref[...],
                             preferred_element_type=jnp.float32)

    return pl.pallas_call(
        body,
        out_shape=jax.ShapeDtypeStruct((S, 2), jnp.float32),
    )(G, W)


def kernel(G_p, Wb, We, k):
    del k
    be = _tc_matvec(G_p, jnp.concatenate([Wb, We], axis=1))
    b = be[:, 0]
    e = be[:, 1]
    oi, oj, vals = _sc_select(b, e)
    return (jnp.concatenate([oi[:, None], oj[:, None]], axis=1), vals)
